# baseline (speedup 1.0000x reference)
import functools

import jax
import jax.numpy as jnp
from jax import lax
from jax.experimental import pallas as pl
from jax.experimental.pallas import tpu as pltpu, tpu_sc as plsc

VOCAB = 1000
B, T = 1024, 50
N_TOK = B * T
LSE_PAD = 1024

NC, NS = 2, 16
NW = NC * NS
TOK_PER_W = N_TOK // NW
BATCH_PER_W = B // NW
NBUF = 2
L = 16
N_GROUPS = TOK_PER_W // L


def _row_lse_body(table_ref, out_ref):
    t = table_ref[...]
    m = jnp.max(t, axis=1, keepdims=True)
    s = jnp.sum(jnp.exp(t - m), axis=1, keepdims=True)
    out_ref[0:VOCAB, :] = m + jnp.log(s)


def _row_lse(table):
    out = pl.pallas_call(
        _row_lse_body,
        out_shape=jax.ShapeDtypeStruct((LSE_PAD, 1), jnp.float32),
    )(table)
    return out.reshape(LSE_PAD)


def _sc_body(idx_hbm, idxp_hbm, tgt_hbm, lse_hbm, table_hbm, tabflat_hbm,
             out_hbm, part_hbm,
             idx_v, idxp_v, tgt_v, comb_v, picked_v, lse_v, rows_v, acc_v,
             gsem, ssem, psem):
    cid = lax.axis_index("c")
    sid = lax.axis_index("s")
    wid = sid * NC + cid
    base = wid * TOK_PER_W
    bbase = wid * BATCH_PER_W

    pltpu.sync_copy(idx_hbm.at[pl.ds(base, TOK_PER_W)], idx_v)
    pltpu.sync_copy(idxp_hbm.at[pl.ds(bbase, BATCH_PER_W)], idxp_v)
    pltpu.sync_copy(tgt_hbm.at[pl.ds(base, TOK_PER_W)], tgt_v)
    pltpu.sync_copy(lse_hbm, lse_v)

    def comb_step(i, _):
        o = i * L
        comb_v[pl.ds(o, L)] = idx_v[pl.ds(o, L)] * VOCAB + tgt_v[pl.ds(o, L)]
        return 0
    lax.fori_loop(0, N_GROUPS, comb_step, 0)

    picked_cp = pltpu.make_async_copy(tabflat_hbm.at[comb_v], picked_v, psem)
    picked_cp.start()

    def gather_desc(g, b):
        return pltpu.make_async_copy(
            table_hbm.at[idxp_v.at[g]], rows_v.at[b], gsem)

    def scatter_desc(g, b):
        return pltpu.make_async_copy(rows_v.at[b], out_hbm.at[bbase + g], ssem)

    gather_desc(0, 0).start()

    def step(g, _):
        b = lax.rem(g, NBUF)
        gather_desc(g, b).wait()

        @pl.when(g >= 1)
        def _():
            scatter_desc(g - 1, 1 - b).wait()

        scatter_desc(g, b).start()

        @pl.when(g + 1 < BATCH_PER_W)
        def _():
            gather_desc(g + 1, 1 - b).start()

        return 0

    lax.fori_loop(0, BATCH_PER_W, step, 0)
    picked_cp.wait()

    def loss_step(i, acc):
        o = i * L
        idxv = idx_v[pl.ds(o, L)]
        lsev = plsc.load_gather(lse_v, [idxv])
        return acc + (lsev - picked_v[pl.ds(o, L)])

    acc = lax.fori_loop(0, N_GROUPS, loss_step, jnp.zeros((L,), jnp.float32))
    scatter_desc(BATCH_PER_W - 1, (BATCH_PER_W - 1) % NBUF).wait()
    acc_v[...] = acc
    pltpu.sync_copy(acc_v, part_hbm.at[wid])


@functools.partial(
    pl.kernel,
    out_type=(
        jax.ShapeDtypeStruct((B, T, VOCAB), jnp.float32),
        jax.ShapeDtypeStruct((NW, L), jnp.float32),
    ),
    mesh=plsc.VectorSubcoreMesh(core_axis_name="c", subcore_axis_name="s"),
    compiler_params=pltpu.CompilerParams(
        needs_layout_passes=False, use_tc_tiling_on_sc=False),
    scratch_types=[
        pltpu.VMEM((TOK_PER_W,), jnp.int32),
        pltpu.VMEM((BATCH_PER_W, T), jnp.int32),
        pltpu.VMEM((TOK_PER_W,), jnp.int32),
        pltpu.VMEM((TOK_PER_W,), jnp.int32),
        pltpu.VMEM((TOK_PER_W,), jnp.float32),
        pltpu.VMEM((LSE_PAD,), jnp.float32),
        pltpu.VMEM((NBUF, T, VOCAB), jnp.float32),
        pltpu.VMEM((L,), jnp.float32),
        pltpu.SemaphoreType.DMA,
        pltpu.SemaphoreType.DMA,
        pltpu.SemaphoreType.DMA,
    ],
)
def _sc_gather_loss(idx_hbm, idxp_hbm, tgt_hbm, lse_hbm, table_hbm,
                    tabflat_hbm, out_hbm, part_hbm,
                    idx_v, idxp_v, tgt_v, comb_v, picked_v, lse_v, rows_v,
                    acc_v, gsem, ssem, psem):
    _sc_body(idx_hbm, idxp_hbm, tgt_hbm, lse_hbm, table_hbm, tabflat_hbm,
             out_hbm, part_hbm,
             idx_v, idxp_v, tgt_v, comb_v, picked_v, lse_v, rows_v, acc_v,
             gsem, ssem, psem)


def kernel(idx, targets, table):
    idx_flat = idx.reshape(N_TOK).astype(jnp.int32)
    tgt_flat = targets.reshape(N_TOK).astype(jnp.int32)
    lse = _row_lse(table)
    tabflat = jnp.concatenate(
        [table.reshape(VOCAB * VOCAB), jnp.zeros((8,), jnp.float32)])
    logits, parts = _sc_gather_loss(
        idx_flat, idx.astype(jnp.int32), tgt_flat, lse, table, tabflat)
    loss = jnp.sum(parts) / jnp.float32(N_TOK)
    return (logits, loss)
